# baseline (speedup 1.0000x reference)
import functools
import math

import jax
import jax.numpy as jnp
from jax import lax
from jax.experimental import pallas as pl
from jax.experimental.pallas import tpu as pltpu
from jax.experimental.pallas import tpu_sc as plsc

S = 2048
H = 32
NB2 = 16
MAX_EXACT = 8
TBL = 2 * S - 1
TBL_PAD = 4112
L = 16
BR = 8
NBLK = S // BR
NBUF = 4

_mesh = plsc.VectorSubcoreMesh(core_axis_name="c", subcore_axis_name="s")


@functools.partial(
    pl.kernel,
    mesh=_mesh,
    compiler_params=pltpu.CompilerParams(
        needs_layout_passes=False,
        use_tc_tiling_on_sc=True,
    ),
    out_type=jax.ShapeDtypeStruct((1, H, S, S), jnp.float32),
    scratch_types=[
        pltpu.VMEM((H * H,), jnp.float32),
        pltpu.VMEM((TBL_PAD,), jnp.int32),
        pltpu.VMEM((TBL_PAD,), jnp.float32),
        pltpu.VMEM((NBUF, BR, S), jnp.float32),
        pltpu.SemaphoreType.DMA,
    ],
)
def _bias_sc(weight_hbm, bucket_hbm, out_hbm, w_v, b_v, vh_v, stage_v, sem):
    h = lax.axis_index("s") * 2 + lax.axis_index("c")

    pltpu.sync_copy(weight_hbm, w_v)
    pltpu.sync_copy(bucket_hbm, b_v)

    hvec = jnp.full((L,), h, dtype=jnp.int32)

    def gather_chunk(k):
        idx = b_v[pl.ds(k, L)] * H + hvec
        vh_v[pl.ds(k, L)] = plsc.load_gather(w_v, [idx])

    plsc.parallel_loop(0, TBL_PAD, L, unroll=8)(gather_chunk)

    def wait_one_block():
        pltpu.make_async_copy(
            stage_v.at[0], out_hbm.at[0, h, pl.ds(0, BR), :], sem
        ).wait()

    def block_body(b, carry):
        @pl.when(b >= NBUF)
        def _():
            wait_one_block()

        buf = b & (NBUF - 1)
        s0 = (S - 1) - b * BR
        for r in range(BR):
            def fill_chunk(c, r=r, buf=buf, sr=s0 - r):
                stage_v[buf, r, pl.ds(c, L)] = vh_v[pl.ds(sr + c, L)]

            plsc.parallel_loop(0, S, L, unroll=16)(fill_chunk)

        row0 = pl.multiple_of(b * BR, BR)
        pltpu.async_copy(
            stage_v.at[buf], out_hbm.at[0, h, pl.ds(row0, BR), :], sem
        )
        return carry

    lax.fori_loop(0, NBLK, block_body, 0)
    for _ in range(NBUF):
        wait_one_block()


def kernel(weight, seq_length):
    del seq_length
    d = jnp.arange(-(S - 1), S, dtype=jnp.int32)
    rel_buckets = (d > 0).astype(jnp.int32) * NB2
    ad = jnp.abs(d)
    is_small = ad < MAX_EXACT
    rp_safe = jnp.maximum(ad, 1)
    large = MAX_EXACT + (
        jnp.log(rp_safe.astype(jnp.float32) / MAX_EXACT)
        / math.log(128 / MAX_EXACT)
        * (NB2 - MAX_EXACT)
    ).astype(jnp.int32)
    large = jnp.minimum(large, NB2 - 1)
    buckets = rel_buckets + jnp.where(is_small, ad, large)
    buckets = jnp.pad(buckets, (0, TBL_PAD - TBL))
    return _bias_sc(weight.reshape(-1), buckets)


# speedup vs baseline: 1.0242x; 1.0242x over previous
import functools
import math

import jax
import jax.numpy as jnp
from jax import lax
from jax.experimental import pallas as pl
from jax.experimental.pallas import tpu as pltpu
from jax.experimental.pallas import tpu_sc as plsc

S = 2048
H = 32
NB2 = 16
MAX_EXACT = 8
TBL = 2 * S - 1
TBL_PAD = 4112
L = 16
BR = 8
NBLK = S // BR
NBUF = 4

_mesh = plsc.VectorSubcoreMesh(core_axis_name="c", subcore_axis_name="s")


@functools.partial(
    pl.kernel,
    mesh=_mesh,
    compiler_params=pltpu.CompilerParams(
        needs_layout_passes=False,
        use_tc_tiling_on_sc=True,
    ),
    out_type=jax.ShapeDtypeStruct((1, H, S, S), jnp.float32),
    scratch_types=[
        pltpu.VMEM((H * H,), jnp.float32),
        pltpu.VMEM((TBL_PAD,), jnp.int32),
        pltpu.VMEM((TBL_PAD,), jnp.float32),
        pltpu.VMEM((NBUF, BR, S), jnp.float32),
        pltpu.SemaphoreType.DMA,
    ],
)
def _bias_sc(weight_hbm, bucket_hbm, out_hbm, w_v, b_v, vh_v, stage_v, sem):
    h = lax.axis_index("s") * 2 + lax.axis_index("c")

    pltpu.sync_copy(weight_hbm, w_v)
    pltpu.sync_copy(bucket_hbm, b_v)

    hvec = jnp.full((L,), h, dtype=jnp.int32)

    def gather_chunk(k):
        idx = b_v[pl.ds(k, L)] * H + hvec
        vh_v[pl.ds(k, L)] = plsc.load_gather(w_v, [idx])

    plsc.parallel_loop(0, TBL_PAD, L, unroll=8)(gather_chunk)

    def wait_one_block():
        pltpu.make_async_copy(
            stage_v.at[0], out_hbm.at[0, h, pl.ds(0, BR), :], sem
        ).wait()

    def block_body(b, carry):
        @pl.when(b >= NBUF)
        def _():
            wait_one_block()

        buf = b & (NBUF - 1)
        s0 = (S - 1) - b * BR
        for r in range(BR):
            def fill_chunk(c, r=r, buf=buf, sr=s0 - r):
                stage_v[buf, r, pl.ds(c, L)] = vh_v[pl.ds(sr + c, L)]

            plsc.parallel_loop(0, S, L, unroll=8)(fill_chunk)

        row0 = pl.multiple_of(b * BR, BR)
        pltpu.async_copy(
            stage_v.at[buf], out_hbm.at[0, h, pl.ds(row0, BR), :], sem
        )
        return carry

    lax.fori_loop(0, NBLK, block_body, 0)
    for _ in range(NBUF):
        wait_one_block()


def kernel(weight, seq_length):
    del seq_length
    d = jnp.arange(-(S - 1), S, dtype=jnp.int32)
    rel_buckets = (d > 0).astype(jnp.int32) * NB2
    ad = jnp.abs(d)
    is_small = ad < MAX_EXACT
    rp_safe = jnp.maximum(ad, 1)
    large = MAX_EXACT + (
        jnp.log(rp_safe.astype(jnp.float32) / MAX_EXACT)
        / math.log(128 / MAX_EXACT)
        * (NB2 - MAX_EXACT)
    ).astype(jnp.int32)
    large = jnp.minimum(large, NB2 - 1)
    buckets = rel_buckets + jnp.where(is_small, ad, large)
    buckets = jnp.pad(buckets, (0, TBL_PAD - TBL))
    return _bias_sc(weight.reshape(-1), buckets)
